# Initial kernel scaffold; baseline (speedup 1.0000x reference)
import jax
import jax.numpy as jnp
from jax.experimental import pallas as pl


def kernel(x, edge_index, batch, W0, b0, g0, be0, Wc1, bc1, g1, be1, Wc2, bc2, g2, be2, Wc3, bc3, g3, be3, W1, b1, W2, b2):
    raise NotImplementedError("write your pallas kernel here")


# speedup vs baseline: 5.9917x; 5.9917x over previous
import functools

import jax
import jax.numpy as jnp
from jax import lax
from jax.experimental import pallas as pl
from jax.experimental.pallas import tpu as pltpu
from jax.experimental.pallas import tpu_sc as plsc

N, E, V, H, C, NB = 10000, 160000, 256, 128, 20, 64

NC, NS = 2, 16
NW = NC * NS
CHUNK = 128
CPW = 40
E_PAD = NW * CPW * CHUNK
NPAD = 10240
STRIPE = NPAD // NS

_mesh = plsc.VectorSubcoreMesh(
    core_axis_name="c", subcore_axis_name="s", num_cores=NC, num_subcores=NS)


EC = 2048
ST = E_PAD // EC


def _sc_scatter(u, src_rows, dst_rows):
    @functools.partial(
        pl.kernel,
        out_type=jax.ShapeDtypeStruct((NC, NPAD, H), jnp.float32),
        mesh=_mesh,
        scratch_types=[
            pltpu.VMEM((CHUNK,), jnp.int32),
            pltpu.VMEM((CHUNK,), jnp.int32),
            pltpu.VMEM((CHUNK, H), jnp.float32),
            pltpu.VMEM_SHARED((NPAD, H), jnp.float32),
            pltpu.SemaphoreType.DMA,
        ],
    )
    def k(u_hbm, src_hbm, dst_hbm, out_hbm, sidx, didx, rows, slab, sem):
        c = lax.axis_index("c")
        s = lax.axis_index("s")
        wid = c * NS + s
        base = s * STRIPE

        @pl.loop(0, CHUNK)
        def _(i):
            for j in range(H // 16):
                rows[i, pl.ds(j * 16, 16)] = jnp.zeros((16,), jnp.float32)

        for t in range(STRIPE // CHUNK):
            pltpu.sync_copy(rows, slab.at[pl.ds(base + t * CHUNK, CHUNK)])
        plsc.subcore_barrier()

        @pl.loop(0, CPW)
        def _(j):
            row = wid * CPW + j
            pltpu.sync_copy(src_hbm.at[row], sidx)
            pltpu.sync_copy(dst_hbm.at[row], didx)
            pltpu.async_copy(u_hbm.at[sidx], rows, sem).wait()
            pltpu.sync_copy(rows, slab.at[didx], add=True)

        plsc.subcore_barrier()
        for t in range(STRIPE // CHUNK):
            pltpu.sync_copy(slab.at[pl.ds(base + t * CHUNK, CHUNK)],
                            out_hbm.at[c, pl.ds(base + t * CHUNK, CHUNK)])

    return k(u, src_rows, dst_rows)


def _bn(y, g, b):
    mu = jnp.mean(y, axis=0)
    var = jnp.mean((y - mu) ** 2, axis=0)
    return g * (y - mu) / jnp.sqrt(var + 1e-5) + b


def _gelu(y):
    return 0.5 * y * (1.0 + lax.erf(y * (2.0 ** -0.5)))


def _tc_deg(dst3):
    def body(dst3_r, dinv_o):
        hi_iota = lax.broadcasted_iota(jnp.int32, (ST, EC), 0)
        lo_iota = lax.broadcasted_iota(jnp.int32, (H, EC), 0)

        def step(i, acc):
            dc = dst3_r[i]
            hi = (dc // H == hi_iota).astype(jnp.bfloat16)
            lo = (dc % H == lo_iota).astype(jnp.bfloat16)
            part = lax.dot_general(
                hi, lo, ((([1]), ([1])), ((), ())),
                preferred_element_type=jnp.float32)
            return acc + part

        deg2d = lax.fori_loop(0, ST, step,
                              jnp.zeros((ST, H), jnp.float32))
        dinv_o[...] = lax.rsqrt(deg2d + 1.0)

    return pl.pallas_call(
        body,
        out_shape=jax.ShapeDtypeStruct((ST, H), jnp.float32),
    )(dst3)


def _tc_a(x, W0, b0, g0, be0, Wc1, bc1, dinv):
    def body(x_r, W0_r, b0_r, g0_r, be0_r, Wc1_r, bc1_r, dinv_r,
             h_o, u1_o):
        y = jnp.dot(x_r[...], W0_r[...],
                    preferred_element_type=jnp.float32) + b0_r[...]
        h = _gelu(_bn(y, g0_r[...], be0_r[...]))
        h_o[...] = h
        p = jnp.dot(h, Wc1_r[...],
                    preferred_element_type=jnp.float32) + bc1_r[...]
        u1_o[...] = p * dinv_r[:N]

    return pl.pallas_call(
        body,
        out_shape=(
            jax.ShapeDtypeStruct((N, H), jnp.float32),
            jax.ShapeDtypeStruct((N, H), jnp.float32),
        ),
    )(x, W0, b0, g0, be0, Wc1, bc1, dinv)


def _tc_b(mparts, u_in, xres, dinv, g, be, Wn, bn_):
    def body(m_r, u_r, xres_r, dinv_r, g_r, be_r, Wn_r, bn_r, h_o, un_o):
        m = m_r[0, :N] + m_r[1, :N]
        dinv = dinv_r[...]
        gcn = (m + u_r[...]) * dinv[:N]
        h = _gelu(_bn(gcn, g_r[...], be_r[...]))
        h_o[...] = h
        p = jnp.dot(xres_r[...] + h, Wn_r[...],
                    preferred_element_type=jnp.float32) + bn_r[...]
        un_o[...] = p * dinv[:N]

    return pl.pallas_call(
        body,
        out_shape=(
            jax.ShapeDtypeStruct((N, H), jnp.float32),
            jax.ShapeDtypeStruct((N, H), jnp.float32),
        ),
    )(mparts, u_in, xres, dinv, g, be, Wn, bn_)


def _tc_d(mparts, u_in, dinv, g, be, batch, W1, b1, W2, b2):
    def body(m_r, u_r, dinv_r, g_r, be_r, batch_r, W1_r, b1_r, W2_r, b2_r,
             out_o):
        m = m_r[0, :N] + m_r[1, :N]
        gcn = (m + u_r[...]) * dinv_r[:N]
        h3 = _gelu(_bn(gcn, g_r[...], be_r[...]))
        seg = lax.broadcasted_iota(jnp.int32, (NB, N), 0)
        onehot = (seg == batch_r[...][None, :]).astype(jnp.float32)
        sums = jnp.dot(onehot, h3, preferred_element_type=jnp.float32)
        cnt = jnp.sum(onehot, axis=1, keepdims=True)
        pooled = sums / jnp.maximum(cnt, 1.0)
        z = _gelu(jnp.dot(pooled, W1_r[...],
                          preferred_element_type=jnp.float32) + b1_r[...])
        logits = jnp.dot(z, W2_r[...],
                         preferred_element_type=jnp.float32) + b2_r[...]
        mx = jnp.max(logits, axis=1, keepdims=True)
        sh = logits - mx
        out_o[...] = sh - jnp.log(jnp.sum(jnp.exp(sh), axis=1, keepdims=True))

    return pl.pallas_call(
        body,
        out_shape=jax.ShapeDtypeStruct((NB, C), jnp.float32),
    )(mparts, u_in, dinv, g, be, batch, W1, b1, W2, b2)


def kernel(x, edge_index, batch, W0, b0, g0, be0, Wc1, bc1, g1, be1,
           Wc2, bc2, g2, be2, Wc3, bc3, g3, be3, W1, b1, W2, b2):
    src = edge_index[0].astype(jnp.int32)
    dst = edge_index[1].astype(jnp.int32)
    pad = E_PAD - E
    src_rows = jnp.concatenate(
        [src, jnp.zeros((pad,), jnp.int32)]).reshape(NW * CPW, CHUNK)
    dst_rows = jnp.concatenate(
        [dst, jnp.full((pad,), N, jnp.int32)]).reshape(NW * CPW, CHUNK)

    dst3 = dst_rows.reshape(ST, 1, EC)
    dinv = _tc_deg(dst3).reshape(NPAD, 1)
    h, u1 = _tc_a(x, W0, b0, g0, be0, Wc1, bc1, dinv)
    m1 = _sc_scatter(u1, src_rows, dst_rows)
    h1, u2 = _tc_b(m1, u1, h, dinv, g1, be1, Wc2, bc2)
    m2 = _sc_scatter(u2, src_rows, dst_rows)
    _, u3 = _tc_b(m2, u2, h1, dinv, g2, be2, Wc3, bc3)
    m3 = _sc_scatter(u3, src_rows, dst_rows)
    return _tc_d(m3, u3, dinv, g3, be3, batch.astype(jnp.int32),
                 W1, b1, W2, b2)


# speedup vs baseline: 7.2023x; 1.2020x over previous
import functools

import jax
import jax.numpy as jnp
from jax import lax
from jax.experimental import pallas as pl
from jax.experimental.pallas import tpu as pltpu
from jax.experimental.pallas import tpu_sc as plsc

N, E, V, H, C, NB = 10000, 160000, 256, 128, 20, 64

NC, NS = 2, 16
NW = NC * NS
CHUNK = 128
CPW = 40
E_PAD = NW * CPW * CHUNK
NPAD = 10240
STRIPE = NPAD // NS

_mesh = plsc.VectorSubcoreMesh(
    core_axis_name="c", subcore_axis_name="s", num_cores=NC, num_subcores=NS)


EC = 2048
ST = E_PAD // EC


def _sc_scatter(u, src_rows, dst_rows):
    @functools.partial(
        pl.kernel,
        out_type=jax.ShapeDtypeStruct((NC, NPAD, H), jnp.float32),
        mesh=_mesh,
        scratch_types=[
            pltpu.VMEM((CPW, CHUNK), jnp.int32),
            pltpu.VMEM((CPW, CHUNK), jnp.int32),
            pltpu.VMEM((CHUNK, H), jnp.float32),
            pltpu.VMEM((CHUNK, H), jnp.float32),
            pltpu.VMEM_SHARED((NPAD, H), jnp.float32),
            pltpu.SemaphoreType.DMA,
            pltpu.SemaphoreType.DMA,
        ],
    )
    def k(u_hbm, src_hbm, dst_hbm, out_hbm, sidx, didx, rows0, rows1,
          slab, sem0, sem1):
        c = lax.axis_index("c")
        s = lax.axis_index("s")
        wid = c * NS + s
        base = s * STRIPE

        @pl.loop(0, CHUNK)
        def _(i):
            for j in range(H // 16):
                rows0[i, pl.ds(j * 16, 16)] = jnp.zeros((16,), jnp.float32)

        for t in range(STRIPE // CHUNK):
            pltpu.sync_copy(rows0, slab.at[pl.ds(base + t * CHUNK, CHUNK)])
        pltpu.sync_copy(src_hbm.at[pl.ds(wid * CPW, CPW)], sidx)
        pltpu.sync_copy(dst_hbm.at[pl.ds(wid * CPW, CPW)], didx)
        plsc.subcore_barrier()

        def gather(j, buf, sem):
            pltpu.async_copy(u_hbm.at[sidx.at[j]], buf, sem)

        def gwait(j, buf, sem):
            pltpu.make_async_copy(u_hbm.at[sidx.at[j]], buf, sem).wait()

        gather(0, rows0, sem0)
        gather(1, rows1, sem1)

        @pl.loop(0, CPW, step=2)
        def _(j):
            gwait(j, rows0, sem0)
            pltpu.sync_copy(rows0, slab.at[didx.at[j]], add=True)

            @pl.when(j + 2 < CPW)
            def _():
                gather(j + 2, rows0, sem0)

            gwait(j + 1, rows1, sem1)
            pltpu.sync_copy(rows1, slab.at[didx.at[j + 1]], add=True)

            @pl.when(j + 3 < CPW)
            def _():
                gather(j + 3, rows1, sem1)

        plsc.subcore_barrier()
        for t in range(STRIPE // CHUNK):
            pltpu.sync_copy(slab.at[pl.ds(base + t * CHUNK, CHUNK)],
                            out_hbm.at[c, pl.ds(base + t * CHUNK, CHUNK)])

    return k(u, src_rows, dst_rows)


def _bn(y, g, b):
    mu = jnp.mean(y, axis=0)
    var = jnp.mean((y - mu) ** 2, axis=0)
    return g * (y - mu) / jnp.sqrt(var + 1e-5) + b


def _gelu(y):
    return 0.5 * y * (1.0 + lax.erf(y * (2.0 ** -0.5)))


def _tc_deg(dst3):
    def body(dst3_r, dinv_o):
        hi_iota = lax.broadcasted_iota(jnp.int32, (ST, EC), 0)
        lo_iota = lax.broadcasted_iota(jnp.int32, (H, EC), 0)

        def step(i, acc):
            dc = dst3_r[i]
            hi = (dc // H == hi_iota).astype(jnp.bfloat16)
            lo = (dc % H == lo_iota).astype(jnp.bfloat16)
            part = lax.dot_general(
                hi, lo, ((([1]), ([1])), ((), ())),
                preferred_element_type=jnp.float32)
            return acc + part

        deg2d = lax.fori_loop(0, ST, step,
                              jnp.zeros((ST, H), jnp.float32))
        dinv_o[...] = lax.rsqrt(deg2d + 1.0)

    return pl.pallas_call(
        body,
        out_shape=jax.ShapeDtypeStruct((ST, H), jnp.float32),
    )(dst3)


def _tc_a(x, W0, b0, g0, be0, Wc1, bc1, dinv):
    def body(x_r, W0_r, b0_r, g0_r, be0_r, Wc1_r, bc1_r, dinv_r,
             h_o, u1_o):
        y = jnp.dot(x_r[...], W0_r[...],
                    preferred_element_type=jnp.float32) + b0_r[...]
        h = _gelu(_bn(y, g0_r[...], be0_r[...]))
        h_o[...] = h
        p = jnp.dot(h, Wc1_r[...],
                    preferred_element_type=jnp.float32) + bc1_r[...]
        u1_o[...] = p * dinv_r[:N]

    return pl.pallas_call(
        body,
        out_shape=(
            jax.ShapeDtypeStruct((N, H), jnp.float32),
            jax.ShapeDtypeStruct((N, H), jnp.float32),
        ),
    )(x, W0, b0, g0, be0, Wc1, bc1, dinv)


def _tc_b(mparts, u_in, xres, dinv, g, be, Wn, bn_):
    def body(m_r, u_r, xres_r, dinv_r, g_r, be_r, Wn_r, bn_r, h_o, un_o):
        m = m_r[0, :N] + m_r[1, :N]
        dinv = dinv_r[...]
        gcn = (m + u_r[...]) * dinv[:N]
        h = _gelu(_bn(gcn, g_r[...], be_r[...]))
        h_o[...] = h
        p = jnp.dot(xres_r[...] + h, Wn_r[...],
                    preferred_element_type=jnp.float32) + bn_r[...]
        un_o[...] = p * dinv[:N]

    return pl.pallas_call(
        body,
        out_shape=(
            jax.ShapeDtypeStruct((N, H), jnp.float32),
            jax.ShapeDtypeStruct((N, H), jnp.float32),
        ),
    )(mparts, u_in, xres, dinv, g, be, Wn, bn_)


def _tc_d(mparts, u_in, dinv, g, be, batch, W1, b1, W2, b2):
    def body(m_r, u_r, dinv_r, g_r, be_r, batch_r, W1_r, b1_r, W2_r, b2_r,
             out_o):
        m = m_r[0, :N] + m_r[1, :N]
        gcn = (m + u_r[...]) * dinv_r[:N]
        h3 = _gelu(_bn(gcn, g_r[...], be_r[...]))
        seg = lax.broadcasted_iota(jnp.int32, (NB, N), 0)
        onehot = (seg == batch_r[...][None, :]).astype(jnp.float32)
        sums = jnp.dot(onehot, h3, preferred_element_type=jnp.float32)
        cnt = jnp.sum(onehot, axis=1, keepdims=True)
        pooled = sums / jnp.maximum(cnt, 1.0)
        z = _gelu(jnp.dot(pooled, W1_r[...],
                          preferred_element_type=jnp.float32) + b1_r[...])
        logits = jnp.dot(z, W2_r[...],
                         preferred_element_type=jnp.float32) + b2_r[...]
        mx = jnp.max(logits, axis=1, keepdims=True)
        sh = logits - mx
        out_o[...] = sh - jnp.log(jnp.sum(jnp.exp(sh), axis=1, keepdims=True))

    return pl.pallas_call(
        body,
        out_shape=jax.ShapeDtypeStruct((NB, C), jnp.float32),
    )(mparts, u_in, dinv, g, be, batch, W1, b1, W2, b2)


def kernel(x, edge_index, batch, W0, b0, g0, be0, Wc1, bc1, g1, be1,
           Wc2, bc2, g2, be2, Wc3, bc3, g3, be3, W1, b1, W2, b2):
    src = edge_index[0].astype(jnp.int32)
    dst = edge_index[1].astype(jnp.int32)
    pad = E_PAD - E
    src_rows = jnp.concatenate(
        [src, jnp.zeros((pad,), jnp.int32)]).reshape(NW * CPW, CHUNK)
    dst_rows = jnp.concatenate(
        [dst, jnp.full((pad,), N, jnp.int32)]).reshape(NW * CPW, CHUNK)

    dst3 = dst_rows.reshape(ST, 1, EC)
    dinv = _tc_deg(dst3).reshape(NPAD, 1)
    h, u1 = _tc_a(x, W0, b0, g0, be0, Wc1, bc1, dinv)
    m1 = _sc_scatter(u1, src_rows, dst_rows)
    h1, u2 = _tc_b(m1, u1, h, dinv, g1, be1, Wc2, bc2)
    m2 = _sc_scatter(u2, src_rows, dst_rows)
    _, u3 = _tc_b(m2, u2, h1, dinv, g2, be2, Wc3, bc3)
    m3 = _sc_scatter(u3, src_rows, dst_rows)
    return _tc_d(m3, u3, dinv, g3, be3, batch.astype(jnp.int32),
                 W1, b1, W2, b2)


# speedup vs baseline: 7.2404x; 1.0053x over previous
import functools

import jax
import jax.numpy as jnp
from jax import lax
from jax.experimental import pallas as pl
from jax.experimental.pallas import tpu as pltpu
from jax.experimental.pallas import tpu_sc as plsc

N, E, V, H, C, NB = 10000, 160000, 256, 128, 20, 64

NC, NS = 2, 16
NW = NC * NS
CHUNK = 128
CPW = 40
E_PAD = NW * CPW * CHUNK
NPAD = 10240
STRIPE = NPAD // NS

_mesh = plsc.VectorSubcoreMesh(
    core_axis_name="c", subcore_axis_name="s", num_cores=NC, num_subcores=NS)


EC = 2048
ST = E_PAD // EC


def _sc_scatter(u, src_rows, dst_rows):
    @functools.partial(
        pl.kernel,
        out_type=jax.ShapeDtypeStruct((NC, NPAD, H), jnp.float32),
        mesh=_mesh,
        scratch_types=[
            pltpu.VMEM((CPW, CHUNK), jnp.int32),
            pltpu.VMEM((CPW, CHUNK), jnp.int32),
            pltpu.VMEM((CHUNK, H), jnp.float32),
            pltpu.VMEM((CHUNK, H), jnp.float32),
            pltpu.VMEM_SHARED((NPAD, H), jnp.float32),
            pltpu.SemaphoreType.DMA,
            pltpu.SemaphoreType.DMA,
        ],
    )
    def k(u_hbm, src_hbm, dst_hbm, out_hbm, sidx, didx, rows0, rows1,
          slab, sem0, sem1):
        c = lax.axis_index("c")
        s = lax.axis_index("s")
        wid = c * NS + s
        base = s * STRIPE

        @pl.loop(0, CHUNK)
        def _(i):
            for j in range(H // 16):
                rows0[i, pl.ds(j * 16, 16)] = jnp.zeros((16,), jnp.float32)

        for t in range(STRIPE // CHUNK):
            pltpu.sync_copy(rows0, slab.at[pl.ds(base + t * CHUNK, CHUNK)])
        pltpu.sync_copy(src_hbm.at[pl.ds(wid * CPW, CPW)], sidx)
        pltpu.sync_copy(dst_hbm.at[pl.ds(wid * CPW, CPW)], didx)
        plsc.subcore_barrier()

        def gather(j, buf, sem):
            pltpu.async_copy(u_hbm.at[sidx.at[j]], buf, sem)

        def gwait(j, buf, sem):
            pltpu.make_async_copy(u_hbm.at[sidx.at[j]], buf, sem).wait()

        gather(0, rows0, sem0)
        gather(1, rows1, sem1)

        @pl.loop(0, CPW, step=2)
        def _(j):
            gwait(j, rows0, sem0)
            pltpu.sync_copy(rows0, slab.at[didx.at[j]], add=True)

            @pl.when(j + 2 < CPW)
            def _():
                gather(j + 2, rows0, sem0)

            gwait(j + 1, rows1, sem1)
            pltpu.sync_copy(rows1, slab.at[didx.at[j + 1]], add=True)

            @pl.when(j + 3 < CPW)
            def _():
                gather(j + 3, rows1, sem1)

        plsc.subcore_barrier()
        for t in range(STRIPE // CHUNK):
            pltpu.sync_copy(slab.at[pl.ds(base + t * CHUNK, CHUNK)],
                            out_hbm.at[c, pl.ds(base + t * CHUNK, CHUNK)])

    return k(u, src_rows, dst_rows)


def _bn(y, g, b):
    mu = jnp.mean(y, axis=0)
    var = jnp.mean((y - mu) ** 2, axis=0)
    return g * (y - mu) / jnp.sqrt(var + 1e-5) + b


def _gelu(y):
    return 0.5 * y * (1.0 + lax.erf(y * (2.0 ** -0.5)))


def _tc_deg(dst3):
    def body(dst3_r, dinv_o):
        hi_iota = lax.broadcasted_iota(jnp.int32, (ST, EC), 0)
        lo_iota = lax.broadcasted_iota(jnp.int32, (H, EC), 0)

        def step(i, acc):
            dc = dst3_r[i]
            hi = (dc // H == hi_iota).astype(jnp.bfloat16)
            lo = (dc % H == lo_iota).astype(jnp.bfloat16)
            part = lax.dot_general(
                hi, lo, ((([1]), ([1])), ((), ())),
                preferred_element_type=jnp.float32)
            return acc + part

        deg2d = lax.fori_loop(0, ST, step,
                              jnp.zeros((ST, H), jnp.float32))
        dinv_o[...] = lax.rsqrt(deg2d + 1.0)

    return pl.pallas_call(
        body,
        out_shape=jax.ShapeDtypeStruct((ST, H), jnp.float32),
    )(dst3)


def _tc_a(x, W0, b0, g0, be0, Wc1, bc1, dinv):
    def body(x_r, W0_r, b0_r, g0_r, be0_r, Wc1_r, bc1_r, dinv_r,
             h_o, u1_o):
        y = jnp.dot(x_r[...], W0_r[...],
                    preferred_element_type=jnp.float32) + b0_r[...]
        h = _gelu(_bn(y, g0_r[...], be0_r[...]))
        h_o[...] = h
        p = jnp.dot(h, Wc1_r[...],
                    preferred_element_type=jnp.float32) + bc1_r[...]
        u1_o[...] = p * dinv_r[:N]

    return pl.pallas_call(
        body,
        out_shape=(
            jax.ShapeDtypeStruct((N, H), jnp.float32),
            jax.ShapeDtypeStruct((N, H), jnp.float32),
        ),
    )(x, W0, b0, g0, be0, Wc1, bc1, dinv)


def _tc_b(mparts, u_in, xres, dinv, g, be, Wn, bn_):
    def body(m_r, u_r, xres_r, dinv_r, g_r, be_r, Wn_r, bn_r, h_o, un_o):
        m = m_r[0, :N] + m_r[1, :N]
        dinv = dinv_r[...]
        gcn = (m + u_r[...]) * dinv[:N]
        h = _gelu(_bn(gcn, g_r[...], be_r[...]))
        h_o[...] = h
        p = jnp.dot(xres_r[...] + h, Wn_r[...],
                    preferred_element_type=jnp.float32) + bn_r[...]
        un_o[...] = p * dinv[:N]

    return pl.pallas_call(
        body,
        out_shape=(
            jax.ShapeDtypeStruct((N, H), jnp.float32),
            jax.ShapeDtypeStruct((N, H), jnp.float32),
        ),
    )(mparts, u_in, xres, dinv, g, be, Wn, bn_)


def _tc_d(mparts, u_in, dinv, g, be, batch, W1, b1, W2, b2):
    def body(m_r, u_r, dinv_r, g_r, be_r, batch_r, W1_r, b1_r, W2_r, b2_r,
             out_o):
        m = m_r[0, :N] + m_r[1, :N]
        gcn = (m + u_r[...]) * dinv_r[:N]
        h3 = _gelu(_bn(gcn, g_r[...], be_r[...]))
        seg = lax.broadcasted_iota(jnp.int32, (NB, N), 0)
        onehot = (seg == batch_r[...][None, :]).astype(jnp.float32)
        sums = jnp.dot(onehot, h3, preferred_element_type=jnp.float32)
        cnt = jnp.sum(onehot, axis=1, keepdims=True)
        pooled = sums / jnp.maximum(cnt, 1.0)
        z = _gelu(jnp.dot(pooled, W1_r[...],
                          preferred_element_type=jnp.float32) + b1_r[...])
        logits = jnp.dot(z, W2_r[...],
                         preferred_element_type=jnp.float32) + b2_r[...]
        mx = jnp.max(logits, axis=1, keepdims=True)
        sh = logits - mx
        out_o[...] = sh - jnp.log(jnp.sum(jnp.exp(sh), axis=1, keepdims=True))

    return pl.pallas_call(
        body,
        out_shape=jax.ShapeDtypeStruct((NB, C), jnp.float32),
    )(mparts, u_in, dinv, g, be, batch, W1, b1, W2, b2)


def kernel(x, edge_index, batch, W0, b0, g0, be0, Wc1, bc1, g1, be1,
           Wc2, bc2, g2, be2, Wc3, bc3, g3, be3, W1, b1, W2, b2):
    src = edge_index[0].astype(jnp.int32)
    dst = edge_index[1].astype(jnp.int32)
    pad = E_PAD - E
    src_rows = jnp.concatenate(
        [src, jnp.zeros((pad,), jnp.int32)]).reshape(NW * CPW, CHUNK)
    pad_dst = N + (jnp.arange(pad, dtype=jnp.int32) % (NPAD - N))
    dst_rows = jnp.concatenate([dst, pad_dst]).reshape(NW * CPW, CHUNK)

    dst3 = dst_rows.reshape(ST, 1, EC)
    dinv = _tc_deg(dst3).reshape(NPAD, 1)
    h, u1 = _tc_a(x, W0, b0, g0, be0, Wc1, bc1, dinv)
    m1 = _sc_scatter(u1, src_rows, dst_rows)
    h1, u2 = _tc_b(m1, u1, h, dinv, g1, be1, Wc2, bc2)
    m2 = _sc_scatter(u2, src_rows, dst_rows)
    _, u3 = _tc_b(m2, u2, h1, dinv, g2, be2, Wc3, bc3)
    m3 = _sc_scatter(u3, src_rows, dst_rows)
    return _tc_d(m3, u3, dinv, g3, be3, batch.astype(jnp.int32),
                 W1, b1, W2, b2)
